# baseline (speedup 1.0000x reference)
import functools

import jax
import jax.numpy as jnp
from jax import lax
from jax.experimental import pallas as pl
from jax.experimental.pallas import tpu as pltpu
from jax.experimental.pallas import tpu_sc as plsc

VOCAB = 100000
C1, C2 = 20000, 60000
D0, D1, D2 = 1024, 256, 128
PROJ = 1024
SCALE = float(PROJ ** 0.5)
T = 8 * 2048

NC, NS = 2, 16
NW = NC * NS
TPW = T // NW

G0, G1, G2 = 64, 128, 128


def _sc_gather(x, emb0, emb1, emb2):
    mesh = plsc.VectorSubcoreMesh(core_axis_name="c", subcore_axis_name="s")

    @functools.partial(
        pl.kernel,
        mesh=mesh,
        out_type=(
            jax.ShapeDtypeStruct((T, D0), jnp.float32),
            jax.ShapeDtypeStruct((T, D1), jnp.float32),
            jax.ShapeDtypeStruct((T, D2), jnp.float32),
        ),
        scratch_types=[
            pltpu.VMEM((TPW,), jnp.int32),
            pltpu.VMEM((TPW,), jnp.int32),
            pltpu.VMEM((TPW,), jnp.int32),
            pltpu.VMEM((TPW,), jnp.int32),
            pltpu.VMEM((G0, D0), jnp.float32),
            pltpu.VMEM((G1, D1), jnp.float32),
            pltpu.VMEM((G2, D2), jnp.float32),
            pltpu.SemaphoreType.DMA,
        ],
    )
    def k(x_hbm, e0_hbm, e1_hbm, e2_hbm, o0_hbm, o1_hbm, o2_hbm,
          x_v, i0_v, i1_v, i2_v, r0_v, r1_v, r2_v, sem):
        wid = lax.axis_index("s") * NC + lax.axis_index("c")
        base = wid * TPW
        pltpu.sync_copy(x_hbm.at[pl.ds(base, TPW)], x_v)
        for j in range(TPW // 16):
            xv = x_v[pl.ds(j * 16, 16)]
            i0_v[pl.ds(j * 16, 16)] = jnp.minimum(xv, C1 - 1)
            i1_v[pl.ds(j * 16, 16)] = jnp.clip(xv - C1, 0, (C2 - C1) - 1)
            i2_v[pl.ds(j * 16, 16)] = jnp.clip(xv - C2, 0, VOCAB - C2)
        for k0 in range(TPW // G0):
            pltpu.async_copy(e0_hbm.at[i0_v.at[pl.ds(k0 * G0, G0)]], r0_v, sem).wait()
            pltpu.sync_copy(r0_v, o0_hbm.at[pl.ds(base + k0 * G0, G0)])
        for k1 in range(TPW // G1):
            pltpu.async_copy(e1_hbm.at[i1_v.at[pl.ds(k1 * G1, G1)]], r1_v, sem).wait()
            pltpu.sync_copy(r1_v, o1_hbm.at[pl.ds(base + k1 * G1, G1)])
        for k2 in range(TPW // G2):
            pltpu.async_copy(e2_hbm.at[i2_v.at[pl.ds(k2 * G2, G2)]], r2_v, sem).wait()
            pltpu.sync_copy(r2_v, o2_hbm.at[pl.ds(base + k2 * G2, G2)])

    return k(x, emb0, emb1, emb2)


BLK = 256


def _tc_body(xb_ref, e0_ref, e1_ref, e2_ref, w0_ref, w1_ref, w2_ref,
             b0_ref, b1_ref, b2_ref, out_ref):
    xv = xb_ref[:, 0:1]
    c1 = xv >= C1
    c2 = xv >= C2
    m0 = jnp.logical_not(c1)
    m1 = jnp.logical_and(c1, jnp.logical_not(c2))
    a0 = jnp.where(m0, e0_ref[...], 0.0)
    a1 = jnp.where(m1, e1_ref[...], 0.0)
    a2 = jnp.where(c2, e2_ref[...], 0.0)
    acc = jnp.dot(a0, w0_ref[...], preferred_element_type=jnp.float32)
    acc += jnp.dot(a1, w1_ref[...], preferred_element_type=jnp.float32)
    acc += jnp.dot(a2, w2_ref[...], preferred_element_type=jnp.float32)
    bias = jnp.where(m0, b0_ref[...], jnp.where(m1, b1_ref[...], b2_ref[...]))
    out_ref[...] = (acc + bias) * SCALE


def _tc_project(xb, e0, e1, e2, W0, b0, W1, b1, W2, b2):
    nblk = T // BLK
    return pl.pallas_call(
        _tc_body,
        grid=(nblk,),
        in_specs=[
            pl.BlockSpec((BLK, 8), lambda i: (i, 0)),
            pl.BlockSpec((BLK, D0), lambda i: (i, 0)),
            pl.BlockSpec((BLK, D1), lambda i: (i, 0)),
            pl.BlockSpec((BLK, D2), lambda i: (i, 0)),
            pl.BlockSpec((D0, PROJ), lambda i: (0, 0)),
            pl.BlockSpec((D1, PROJ), lambda i: (0, 0)),
            pl.BlockSpec((D2, PROJ), lambda i: (0, 0)),
            pl.BlockSpec((1, PROJ), lambda i: (0, 0)),
            pl.BlockSpec((1, PROJ), lambda i: (0, 0)),
            pl.BlockSpec((1, PROJ), lambda i: (0, 0)),
        ],
        out_specs=pl.BlockSpec((BLK, PROJ), lambda i: (i, 0)),
        out_shape=jax.ShapeDtypeStruct((T, PROJ), jnp.float32),
    )(xb, e0, e1, e2, W0, W1, W2, b0, b1, b2)


def kernel(x, emb0, emb1, emb2, W0, b0, W1, b1, W2, b2):
    flat_x = x.reshape(-1)
    emb2p = jnp.pad(emb2, ((0, 0), (0, D2 - emb2.shape[1])))
    W2 = jnp.pad(W2, ((0, D2 - W2.shape[0]), (0, 0)))
    e0, e1, e2 = _sc_gather(flat_x, emb0, emb1, emb2p)
    xb = jnp.broadcast_to(flat_x[:, None], (T, 8))
    out = _tc_project(xb, e0, e1, e2,
                      W0, b0[None, :], W1, b1[None, :], W2, b2[None, :])
    return out.reshape(x.shape + (PROJ,))
